# Initial kernel scaffold; baseline (speedup 1.0000x reference)
import jax
import jax.numpy as jnp
from jax.experimental import pallas as pl


def kernel(mh_output, W, b):
    raise NotImplementedError("write your pallas kernel here")


# speedup vs baseline: 1.6824x; 1.6824x over previous
import jax
import jax.numpy as jnp
from jax.experimental import pallas as pl

N_EMBED = 768
NUM_EXPERTS = 64
NEG_INF = float("-inf")


def _router_body(x_ref, wt_ref, b_ref, out_ref, idx_ref):
    x = x_ref[...]
    logits = jnp.dot(x, wt_ref[...],
                     preferred_element_type=jnp.float32) + b_ref[...]
    eiota = jax.lax.broadcasted_iota(jnp.int32, logits.shape, 1)
    i1 = jnp.argmax(logits, axis=-1)
    is1 = eiota == i1[:, None]
    m1 = jnp.max(logits, axis=-1, keepdims=True)
    masked = jnp.where(is1, NEG_INF, logits)
    i2 = jnp.argmax(masked, axis=-1)
    is2 = eiota == i2[:, None]
    m2 = jnp.max(masked, axis=-1, keepdims=True)
    e = jnp.exp(m2 - m1)
    denom = 1.0 + e
    p1 = 1.0 / denom
    p2 = e / denom
    out_ref[...] = jnp.where(is1, p1, 0.0) + jnp.where(is2, p2, 0.0)
    idx_ref[...] = jnp.concatenate([i1[:, None], i2[:, None]], axis=-1)


def kernel(mh_output, W, b):
    B, S, D = mh_output.shape
    T = B * S
    x = mh_output.reshape(T, D)
    wt = W.T
    b2 = b.reshape(1, NUM_EXPERTS)

    BT = 512
    grid = (T // BT,)
    out, idx = pl.pallas_call(
        _router_body,
        grid=grid,
        in_specs=[
            pl.BlockSpec((BT, D), lambda i: (i, 0)),
            pl.BlockSpec((D, NUM_EXPERTS), lambda i: (0, 0)),
            pl.BlockSpec((1, NUM_EXPERTS), lambda i: (0, 0)),
        ],
        out_specs=[
            pl.BlockSpec((BT, NUM_EXPERTS), lambda i: (i, 0)),
            pl.BlockSpec((BT, 2), lambda i: (i, 0)),
        ],
        out_shape=[
            jax.ShapeDtypeStruct((T, NUM_EXPERTS), jnp.float32),
            jax.ShapeDtypeStruct((T, 2), jnp.int32),
        ],
    )(x, wt, b2)
    return out.reshape(B, S, NUM_EXPERTS), idx.reshape(B, S, 2)


# speedup vs baseline: 2.1064x; 1.2520x over previous
import jax
import jax.numpy as jnp
from jax.experimental import pallas as pl

N_EMBED = 768
NUM_EXPERTS = 64
NEG_INF = float("-inf")


def _router_body(x_ref, wt_ref, b_ref, out_ref, idx_ref):
    x = x_ref[...]
    logits = jnp.dot(x, wt_ref[...],
                     preferred_element_type=jnp.float32) + b_ref[...]
    eiota = jax.lax.broadcasted_iota(jnp.int32, logits.shape, 1)
    i1 = jnp.argmax(logits, axis=-1)
    is1 = eiota == i1[:, None]
    m1 = jnp.max(logits, axis=-1, keepdims=True)
    masked = jnp.where(is1, NEG_INF, logits)
    i2 = jnp.argmax(masked, axis=-1)
    is2 = eiota == i2[:, None]
    m2 = jnp.max(masked, axis=-1, keepdims=True)
    e = jnp.exp(m2 - m1)
    denom = 1.0 + e
    p1 = 1.0 / denom
    p2 = e / denom
    out_ref[...] = jnp.where(is1, p1, 0.0) + jnp.where(is2, p2, 0.0)
    idx_ref[...] = jnp.concatenate([i1[:, None], i2[:, None]], axis=-1)


def kernel(mh_output, W, b):
    B, S, D = mh_output.shape
    T = B * S
    x = mh_output.reshape(T, D)
    wt = W.T
    b2 = b.reshape(1, NUM_EXPERTS)

    BT = 1024
    grid = (T // BT,)
    out, idx = pl.pallas_call(
        _router_body,
        grid=grid,
        in_specs=[
            pl.BlockSpec((BT, D), lambda i: (i, 0)),
            pl.BlockSpec((D, NUM_EXPERTS), lambda i: (0, 0)),
            pl.BlockSpec((1, NUM_EXPERTS), lambda i: (0, 0)),
        ],
        out_specs=[
            pl.BlockSpec((BT, NUM_EXPERTS), lambda i: (i, 0)),
            pl.BlockSpec((BT, 2), lambda i: (i, 0)),
        ],
        out_shape=[
            jax.ShapeDtypeStruct((T, NUM_EXPERTS), jnp.float32),
            jax.ShapeDtypeStruct((T, 2), jnp.int32),
        ],
    )(x, wt, b2)
    return out.reshape(B, S, NUM_EXPERTS), idx.reshape(B, S, 2)


# speedup vs baseline: 2.3395x; 1.1106x over previous
import jax
import jax.numpy as jnp
from jax.experimental import pallas as pl

N_EMBED = 768
NUM_EXPERTS = 64
NEG_INF = float("-inf")


def _router_body(x_ref, wt_ref, b_ref, out_ref, idx_ref):
    x = x_ref[...]
    logits = jnp.dot(x, wt_ref[...],
                     preferred_element_type=jnp.float32) + b_ref[...]
    eiota = jax.lax.broadcasted_iota(jnp.int32, logits.shape, 1)
    i1 = jnp.argmax(logits, axis=-1)
    is1 = eiota == i1[:, None]
    m1 = jnp.max(logits, axis=-1, keepdims=True)
    masked = jnp.where(is1, NEG_INF, logits)
    i2 = jnp.argmax(masked, axis=-1)
    is2 = eiota == i2[:, None]
    m2 = jnp.max(masked, axis=-1, keepdims=True)
    e = jnp.exp(m2 - m1)
    denom = 1.0 + e
    p1 = 1.0 / denom
    p2 = e / denom
    out_ref[...] = jnp.where(is1, p1, 0.0) + jnp.where(is2, p2, 0.0)
    idx_ref[...] = jnp.concatenate([i1[:, None], i2[:, None]], axis=-1)


def kernel(mh_output, W, b):
    B, S, D = mh_output.shape
    T = B * S
    x = mh_output.reshape(T, D)
    wt = W.T
    b2 = b.reshape(1, NUM_EXPERTS)

    BT = 2048
    grid = (T // BT,)
    out, idx = pl.pallas_call(
        _router_body,
        grid=grid,
        in_specs=[
            pl.BlockSpec((BT, D), lambda i: (i, 0)),
            pl.BlockSpec((D, NUM_EXPERTS), lambda i: (0, 0)),
            pl.BlockSpec((1, NUM_EXPERTS), lambda i: (0, 0)),
        ],
        out_specs=[
            pl.BlockSpec((BT, NUM_EXPERTS), lambda i: (i, 0)),
            pl.BlockSpec((BT, 2), lambda i: (i, 0)),
        ],
        out_shape=[
            jax.ShapeDtypeStruct((T, NUM_EXPERTS), jnp.float32),
            jax.ShapeDtypeStruct((T, 2), jnp.int32),
        ],
    )(x, wt, b2)
    return out.reshape(B, S, NUM_EXPERTS), idx.reshape(B, S, 2)


# speedup vs baseline: 2.4601x; 1.0516x over previous
import jax
import jax.numpy as jnp
from jax.experimental import pallas as pl

N_EMBED = 768
NUM_EXPERTS = 64
NEG_INF = float("-inf")


def _router_body(x_ref, wt_ref, b_ref, out_ref, idx_ref):
    x = x_ref[...]
    logits = jnp.dot(x, wt_ref[...],
                     preferred_element_type=jnp.float32) + b_ref[...]
    eiota = jax.lax.broadcasted_iota(jnp.int32, logits.shape, 1)
    i1 = jnp.argmax(logits, axis=-1)
    is1 = eiota == i1[:, None]
    m1 = jnp.max(logits, axis=-1, keepdims=True)
    masked = jnp.where(is1, NEG_INF, logits)
    i2 = jnp.argmax(masked, axis=-1)
    is2 = eiota == i2[:, None]
    m2 = jnp.max(masked, axis=-1, keepdims=True)
    e = jnp.exp(m2 - m1)
    denom = 1.0 + e
    p1 = 1.0 / denom
    p2 = e / denom
    out_ref[...] = jnp.where(is1, p1, 0.0) + jnp.where(is2, p2, 0.0)
    idx_ref[...] = jnp.concatenate([i1[:, None], i2[:, None]], axis=-1)


def kernel(mh_output, W, b):
    B, S, D = mh_output.shape
    T = B * S
    x = mh_output.reshape(T, D)
    wt = W.T
    b2 = b.reshape(1, NUM_EXPERTS)

    BT = 4096
    grid = (T // BT,)
    out, idx = pl.pallas_call(
        _router_body,
        grid=grid,
        in_specs=[
            pl.BlockSpec((BT, D), lambda i: (i, 0)),
            pl.BlockSpec((D, NUM_EXPERTS), lambda i: (0, 0)),
            pl.BlockSpec((1, NUM_EXPERTS), lambda i: (0, 0)),
        ],
        out_specs=[
            pl.BlockSpec((BT, NUM_EXPERTS), lambda i: (i, 0)),
            pl.BlockSpec((BT, 2), lambda i: (i, 0)),
        ],
        out_shape=[
            jax.ShapeDtypeStruct((T, NUM_EXPERTS), jnp.float32),
            jax.ShapeDtypeStruct((T, 2), jnp.int32),
        ],
    )(x, wt, b2)
    return out.reshape(B, S, NUM_EXPERTS), idx.reshape(B, S, 2)


# speedup vs baseline: 2.4611x; 1.0004x over previous
import jax
import jax.numpy as jnp
from jax.experimental import pallas as pl
from jax.experimental.pallas import tpu as pltpu

N_EMBED = 768
NUM_EXPERTS = 64
NEG_INF = float("-inf")


def _router_body(x_ref, wt_ref, b_ref, out_ref, idx_ref):
    x = x_ref[...]
    logits = jnp.dot(x, wt_ref[...],
                     preferred_element_type=jnp.float32) + b_ref[...]
    eiota = jax.lax.broadcasted_iota(jnp.int32, logits.shape, 1)
    i1 = jnp.argmax(logits, axis=-1)
    is1 = eiota == i1[:, None]
    m1 = jnp.max(logits, axis=-1, keepdims=True)
    masked = jnp.where(is1, NEG_INF, logits)
    i2 = jnp.argmax(masked, axis=-1)
    is2 = eiota == i2[:, None]
    m2 = jnp.max(masked, axis=-1, keepdims=True)
    e = jnp.exp(m2 - m1)
    denom = 1.0 + e
    p1 = 1.0 / denom
    p2 = e / denom
    out_ref[...] = jnp.where(is1, p1, 0.0) + jnp.where(is2, p2, 0.0)
    idx_ref[...] = jnp.concatenate([i1[:, None], i2[:, None]], axis=-1)


def kernel(mh_output, W, b):
    B, S, D = mh_output.shape
    T = B * S
    x = mh_output.reshape(T, D)
    wt = W.T
    b2 = b.reshape(1, NUM_EXPERTS)

    BT = 4096
    grid = (T // BT,)
    out, idx = pl.pallas_call(
        _router_body,
        grid=grid,
        in_specs=[
            pl.BlockSpec((BT, D), lambda i: (i, 0)),
            pl.BlockSpec((D, NUM_EXPERTS), lambda i: (0, 0)),
            pl.BlockSpec((1, NUM_EXPERTS), lambda i: (0, 0)),
        ],
        out_specs=[
            pl.BlockSpec((BT, NUM_EXPERTS), lambda i: (i, 0)),
            pl.BlockSpec((BT, 2), lambda i: (i, 0)),
        ],
        out_shape=[
            jax.ShapeDtypeStruct((T, NUM_EXPERTS), jnp.float32),
            jax.ShapeDtypeStruct((T, 2), jnp.int32),
        ],
        compiler_params=pltpu.CompilerParams(
            dimension_semantics=("parallel",),
        ),
    )(x, wt, b2)
    return out.reshape(B, S, NUM_EXPERTS), idx.reshape(B, S, 2)


# speedup vs baseline: 2.5265x; 1.0266x over previous
import jax
import jax.numpy as jnp
from jax.experimental import pallas as pl
from jax.experimental.pallas import tpu as pltpu

N_EMBED = 768
NUM_EXPERTS = 64
NEG_INF = float("-inf")


def _router_body(x_ref, w_ref, b_ref, out_ref, idx_ref):
    x = x_ref[...]
    logits = jax.lax.dot_general(
        x, w_ref[...], (((1,), (1,)), ((), ())),
        preferred_element_type=jnp.float32) + b_ref[...]
    eiota = jax.lax.broadcasted_iota(jnp.int32, logits.shape, 1)
    i1 = jnp.argmax(logits, axis=-1)
    is1 = eiota == i1[:, None]
    m1 = jnp.max(logits, axis=-1, keepdims=True)
    masked = jnp.where(is1, NEG_INF, logits)
    i2 = jnp.argmax(masked, axis=-1)
    is2 = eiota == i2[:, None]
    m2 = jnp.max(masked, axis=-1, keepdims=True)
    e = jnp.exp(m2 - m1)
    denom = 1.0 + e
    p1 = 1.0 / denom
    p2 = e / denom
    out_ref[...] = jnp.where(is1, p1, 0.0) + jnp.where(is2, p2, 0.0)
    idx_ref[...] = jnp.concatenate([i1[:, None], i2[:, None]], axis=-1)


def kernel(mh_output, W, b):
    B, S, D = mh_output.shape
    T = B * S
    x = mh_output.reshape(T, D)
    b2 = b.reshape(1, NUM_EXPERTS)

    BT = 4096
    grid = (T // BT,)
    out, idx = pl.pallas_call(
        _router_body,
        grid=grid,
        in_specs=[
            pl.BlockSpec((BT, D), lambda i: (i, 0)),
            pl.BlockSpec((NUM_EXPERTS, D), lambda i: (0, 0)),
            pl.BlockSpec((1, NUM_EXPERTS), lambda i: (0, 0)),
        ],
        out_specs=[
            pl.BlockSpec((BT, NUM_EXPERTS), lambda i: (i, 0)),
            pl.BlockSpec((BT, 2), lambda i: (i, 0)),
        ],
        out_shape=[
            jax.ShapeDtypeStruct((T, NUM_EXPERTS), jnp.float32),
            jax.ShapeDtypeStruct((T, 2), jnp.int32),
        ],
        compiler_params=pltpu.CompilerParams(
            dimension_semantics=("parallel",),
        ),
    )(x, W, b2)
    return out.reshape(B, S, NUM_EXPERTS), idx.reshape(B, S, 2)


# speedup vs baseline: 2.7435x; 1.0859x over previous
import jax
import jax.numpy as jnp
from jax.experimental import pallas as pl
from jax.experimental.pallas import tpu as pltpu

N_EMBED = 768
NUM_EXPERTS = 64
NEG_INF = float("-inf")


def _router_body(x_ref, w_ref, b_ref, out_ref, idx_ref):
    x = x_ref[0]
    logits = jax.lax.dot_general(
        x, w_ref[...], (((1,), (1,)), ((), ())),
        preferred_element_type=jnp.float32) + b_ref[...]
    eiota = jax.lax.broadcasted_iota(jnp.int32, logits.shape, 1)
    i1 = jnp.argmax(logits, axis=-1)
    is1 = eiota == i1[:, None]
    m1 = jnp.max(logits, axis=-1, keepdims=True)
    masked = jnp.where(is1, NEG_INF, logits)
    i2 = jnp.argmax(masked, axis=-1)
    is2 = eiota == i2[:, None]
    m2 = jnp.max(masked, axis=-1, keepdims=True)
    e = jnp.exp(m2 - m1)
    denom = 1.0 + e
    p1 = 1.0 / denom
    p2 = e / denom
    out_ref[0] = jnp.where(is1, p1, 0.0) + jnp.where(is2, p2, 0.0)
    idx_ref[0] = jnp.concatenate([i1[:, None], i2[:, None]], axis=-1)


def kernel(mh_output, W, b):
    B, S, D = mh_output.shape
    b2 = b.reshape(1, NUM_EXPERTS)

    BT = 4096
    grid = (B, S // BT)
    out, idx = pl.pallas_call(
        _router_body,
        grid=grid,
        in_specs=[
            pl.BlockSpec((1, BT, D), lambda i, j: (i, j, 0)),
            pl.BlockSpec((NUM_EXPERTS, D), lambda i, j: (0, 0)),
            pl.BlockSpec((1, NUM_EXPERTS), lambda i, j: (0, 0)),
        ],
        out_specs=[
            pl.BlockSpec((1, BT, NUM_EXPERTS), lambda i, j: (i, j, 0)),
            pl.BlockSpec((1, BT, 2), lambda i, j: (i, j, 0)),
        ],
        out_shape=[
            jax.ShapeDtypeStruct((B, S, NUM_EXPERTS), jnp.float32),
            jax.ShapeDtypeStruct((B, S, 2), jnp.int32),
        ],
        compiler_params=pltpu.CompilerParams(
            dimension_semantics=("parallel", "parallel"),
        ),
    )(mh_output, W, b2)
    return out, idx
